# baseline (speedup 1.0000x reference)
import functools

import jax
import jax.numpy as jnp
from jax import lax
from jax.experimental import pallas as pl
from jax.experimental.pallas import tpu as pltpu
from jax.experimental.pallas import tpu_sc as plsc

N = 10000
E = 320000
D = 128

NC = 2
NS = 16
NW = NC * NS
CHUNK = 128
EPW = 10240
NCHUNK = EPW // CHUNK
SB = 8
NSB = NCHUNK // SB
EPAD = NW * EPW - E
NP = 10240
ROWS_PER_TILE = NP // NS


def _sc_agg_kernel(*args, with_cnt):
    if with_cnt:
        (table, srcr, dstr, zrows, zcnt, ones_hbm, pout, cout,
         accum, cnts, srcv, dstv, rows, onesv) = args
    else:
        (table, srcr, dstr, zrows, pout,
         accum, srcv, dstv, rows) = args
        cnts = onesv = cout = zcnt = ones_hbm = None

    c = lax.axis_index("c")
    s = lax.axis_index("s")
    w = c * NS + s
    sl = pl.ds(s * ROWS_PER_TILE, ROWS_PER_TILE)

    pltpu.sync_copy(zrows, accum.at[sl])
    if with_cnt:
        pltpu.sync_copy(zcnt, cnts.at[sl])
        pltpu.sync_copy(ones_hbm, onesv)
    pltpu.sync_copy(srcr.at[w], srcv)
    pltpu.sync_copy(dstr.at[w], dstv)
    plsc.subcore_barrier()

    def chunk(j, carry):
        pltpu.sync_copy(table.at[srcv.at[j]], rows)
        pltpu.sync_copy(rows, accum.at[dstv.at[j]], add=True)
        if with_cnt:
            pltpu.sync_copy(onesv, cnts.at[dstv.at[j]], add=True)
        return carry

    lax.fori_loop(0, NCHUNK, chunk, 0)
    plsc.subcore_barrier()

    pltpu.sync_copy(accum.at[sl], pout.at[c, sl])
    if with_cnt:
        pltpu.sync_copy(cnts.at[sl], cout.at[c, sl])


@functools.partial(jax.jit, static_argnames=("with_cnt",))
def _sc_agg(table, srcr, dstr, with_cnt):
    mesh = plsc.VectorSubcoreMesh(core_axis_name="c", subcore_axis_name="s")
    out_type = [jax.ShapeDtypeStruct((NC, NP, D), jnp.float32)]
    scratch = [pltpu.VMEM_SHARED((NP, D), jnp.float32)]
    if with_cnt:
        out_type.append(jax.ShapeDtypeStruct((NC, NP), jnp.float32))
        scratch.append(pltpu.VMEM_SHARED((NP,), jnp.float32))
    scratch += [
        pltpu.VMEM((NCHUNK, CHUNK), jnp.int32),
        pltpu.VMEM((NCHUNK, CHUNK), jnp.int32),
        pltpu.VMEM((CHUNK, D), jnp.float32),
    ]
    if with_cnt:
        scratch.append(pltpu.VMEM((CHUNK,), jnp.float32))

    inputs = [table, srcr, dstr, jnp.zeros((ROWS_PER_TILE, D), jnp.float32)]
    if with_cnt:
        inputs += [jnp.zeros((ROWS_PER_TILE,), jnp.float32),
                   jnp.ones((CHUNK,), jnp.float32)]
    f = functools.partial(
        pl.kernel,
        out_type=out_type,
        mesh=mesh,
        scratch_types=scratch,
    )(functools.partial(_sc_agg_kernel, with_cnt=with_cnt))
    return f(*inputs)


def _dense_kernel(p_ref, c_ref, x_ref, wn_ref, wr_ref, b_ref, o_ref, *, relu):
    agg = p_ref[0] + p_ref[1]
    cnt = c_ref[0, :, 0] + c_ref[1, :, 0]
    mean = agg * (1.0 / jnp.maximum(cnt, 1.0))[:, None]
    out = (jnp.dot(mean, wn_ref[...], precision=lax.Precision.HIGHEST)
           + jnp.dot(x_ref[...], wr_ref[...], precision=lax.Precision.HIGHEST)
           + b_ref[...][None, :])
    if relu:
        out = jnp.maximum(out, 0.0)
    o_ref[...] = out


def _dense(p, c, x, Wn, Wr, b, relu):
    BR = 1000
    grid = (N // BR,)
    return pl.pallas_call(
        functools.partial(_dense_kernel, relu=relu),
        grid=grid,
        in_specs=[
            pl.BlockSpec((NC, BR, D), lambda i: (0, i, 0)),
            pl.BlockSpec((NC, BR, 1), lambda i: (0, i, 0)),
            pl.BlockSpec((BR, D), lambda i: (i, 0)),
            pl.BlockSpec((D, D), lambda i: (0, 0)),
            pl.BlockSpec((D, D), lambda i: (0, 0)),
            pl.BlockSpec((D,), lambda i: (0,)),
        ],
        out_specs=pl.BlockSpec((BR, D), lambda i: (i, 0)),
        out_shape=jax.ShapeDtypeStruct((N, D), jnp.float32),
    )(p, c, x, Wn, Wr, b)


def kernel(x, edge_index, Wn1, Wr1, b1, Wn2, Wr2, b2):
    src = jnp.concatenate(
        [edge_index[0].astype(jnp.int32), jnp.zeros((EPAD,), jnp.int32)]
    ).reshape(NW, NCHUNK, CHUNK)
    pad_dst = N + jnp.arange(EPAD, dtype=jnp.int32) % (NP - N)
    dst = jnp.concatenate(
        [edge_index[1].astype(jnp.int32), pad_dst]
    ).reshape(NW, NCHUNK, CHUNK)

    p1, c1 = _sc_agg(x, src, dst, with_cnt=True)
    c1 = c1[..., None]
    h = _dense(p1, c1, x, Wn1, Wr1, b1, relu=True)
    p2 = _sc_agg(h, src, dst, with_cnt=False)
    if isinstance(p2, (list, tuple)):
        p2 = p2[0]
    out = _dense(p2, c1, h, Wn2, Wr2, b2, relu=False)
    return out


# speedup vs baseline: 2.8065x; 2.8065x over previous
import functools

import jax
import jax.numpy as jnp
from jax import lax
from jax.experimental import pallas as pl
from jax.experimental.pallas import tpu as pltpu
from jax.experimental.pallas import tpu_sc as plsc

N = 10000
E = 320000
D = 128

NC = 2
NS = 16
NW = NC * NS
CHUNK = 125
EPW = E // NW
NCHUNK = EPW // CHUNK
NP = 10240
ROWS_PER_TILE = NP // NS


def _sc_agg_kernel(*args, with_cnt):
    if with_cnt:
        (table, srcr, dstr, zrows, zcnt, ones_hbm, pout, cout,
         accum, cnts, srcv, dstv, rows, onesv) = args
    else:
        (table, srcr, dstr, zrows, pout,
         accum, srcv, dstv, rows) = args
        cnts = onesv = cout = zcnt = ones_hbm = None

    c = lax.axis_index("c")
    s = lax.axis_index("s")
    w = c * NS + s
    sl = pl.ds(s * ROWS_PER_TILE, ROWS_PER_TILE)

    pltpu.sync_copy(zrows, accum.at[sl])
    if with_cnt:
        pltpu.sync_copy(zcnt, cnts.at[sl])
        pltpu.sync_copy(ones_hbm, onesv)
    pltpu.sync_copy(srcr.at[w], srcv)
    pltpu.sync_copy(dstr.at[w], dstv)
    plsc.subcore_barrier()

    def chunk(j, carry):
        pltpu.sync_copy(table.at[srcv.at[j]], rows)
        pltpu.sync_copy(rows, accum.at[dstv.at[j]], add=True)
        if with_cnt:
            pltpu.sync_copy(onesv, cnts.at[dstv.at[j]], add=True)
        return carry

    lax.fori_loop(0, NCHUNK, chunk, 0)
    plsc.subcore_barrier()

    pltpu.sync_copy(accum.at[sl], pout.at[c, sl])
    if with_cnt:
        pltpu.sync_copy(cnts.at[sl], cout.at[c, sl])


@functools.partial(jax.jit, static_argnames=("with_cnt",))
def _sc_agg(table, srcr, dstr, with_cnt):
    mesh = plsc.VectorSubcoreMesh(core_axis_name="c", subcore_axis_name="s")
    out_type = [jax.ShapeDtypeStruct((NC, NP, D), jnp.float32)]
    scratch = [pltpu.VMEM_SHARED((NP, D), jnp.float32)]
    if with_cnt:
        out_type.append(jax.ShapeDtypeStruct((NC, NP), jnp.float32))
        scratch.append(pltpu.VMEM_SHARED((NP,), jnp.float32))
    scratch += [
        pltpu.VMEM((NCHUNK, CHUNK), jnp.int32),
        pltpu.VMEM((NCHUNK, CHUNK), jnp.int32),
        pltpu.VMEM((CHUNK, D), jnp.float32),
    ]
    if with_cnt:
        scratch.append(pltpu.VMEM((CHUNK,), jnp.float32))

    inputs = [table, srcr, dstr, jnp.zeros((ROWS_PER_TILE, D), jnp.float32)]
    if with_cnt:
        inputs += [jnp.zeros((ROWS_PER_TILE,), jnp.float32),
                   jnp.ones((CHUNK,), jnp.float32)]
    f = functools.partial(
        pl.kernel,
        out_type=out_type,
        mesh=mesh,
        scratch_types=scratch,
    )(functools.partial(_sc_agg_kernel, with_cnt=with_cnt))
    return f(*inputs)


def _dense_kernel(p_ref, c_ref, x_ref, wn_ref, wr_ref, b_ref, o_ref, *, relu):
    agg = p_ref[0] + p_ref[1]
    cnt = c_ref[0, :, 0] + c_ref[1, :, 0]
    mean = agg * (1.0 / jnp.maximum(cnt, 1.0))[:, None]
    out = (jnp.dot(mean, wn_ref[...], precision=lax.Precision.HIGHEST)
           + jnp.dot(x_ref[...], wr_ref[...], precision=lax.Precision.HIGHEST)
           + b_ref[...][None, :])
    if relu:
        out = jnp.maximum(out, 0.0)
    o_ref[...] = out


def _dense(p, c, x, Wn, Wr, b, relu):
    BR = 1000
    grid = (N // BR,)
    return pl.pallas_call(
        functools.partial(_dense_kernel, relu=relu),
        grid=grid,
        in_specs=[
            pl.BlockSpec((NC, BR, D), lambda i: (0, i, 0)),
            pl.BlockSpec((NC, BR, 1), lambda i: (0, i, 0)),
            pl.BlockSpec((BR, D), lambda i: (i, 0)),
            pl.BlockSpec((D, D), lambda i: (0, 0)),
            pl.BlockSpec((D, D), lambda i: (0, 0)),
            pl.BlockSpec((D,), lambda i: (0,)),
        ],
        out_specs=pl.BlockSpec((BR, D), lambda i: (i, 0)),
        out_shape=jax.ShapeDtypeStruct((N, D), jnp.float32),
    )(p, c, x, Wn, Wr, b)


def kernel(x, edge_index, Wn1, Wr1, b1, Wn2, Wr2, b2):
    src = edge_index[0].astype(jnp.int32).reshape(NW, NCHUNK, CHUNK)
    dst = edge_index[1].astype(jnp.int32).reshape(NW, NCHUNK, CHUNK)

    p1, c1 = _sc_agg(x, src, dst, with_cnt=True)
    c1 = c1[..., None]
    h = _dense(p1, c1, x, Wn1, Wr1, b1, relu=True)
    p2 = _sc_agg(h, src, dst, with_cnt=False)
    if isinstance(p2, (list, tuple)):
        p2 = p2[0]
    out = _dense(p2, c1, h, Wn2, Wr2, b2, relu=False)
    return out


# speedup vs baseline: 3.5980x; 1.2820x over previous
import functools

import jax
import jax.numpy as jnp
from jax import lax
from jax.experimental import pallas as pl
from jax.experimental.pallas import tpu as pltpu
from jax.experimental.pallas import tpu_sc as plsc

N = 10000
E = 320000
D = 128

NC = 2
NS = 16
NW = NC * NS
CHUNK = 125
EPW = E // NW
NCHUNK = EPW // CHUNK
SB = 8
NSB = NCHUNK // SB
NP = 10240
ROWS_PER_TILE = NP // NS


def _sc_agg_kernel(*args, with_cnt):
    if with_cnt:
        (table, srcr, dstr, zrows, zcnt, ones_hbm, pout, cout,
         accum, cnts, srcv, dstv, rows, onesv, sem, isem) = args
    else:
        (table, srcr, dstr, zrows, pout,
         accum, srcv, dstv, rows, sem, isem) = args
        cnts = onesv = cout = zcnt = ones_hbm = None

    c = lax.axis_index("c")
    s = lax.axis_index("s")
    w = c * NS + s
    sl = pl.ds(s * ROWS_PER_TILE, ROWS_PER_TILE)

    pltpu.sync_copy(zrows, accum.at[sl])
    if with_cnt:
        pltpu.sync_copy(zcnt, cnts.at[sl])
        pltpu.sync_copy(ones_hbm, onesv)
    pltpu.sync_copy(srcr.at[w], srcv)
    pltpu.sync_copy(dstr.at[w, pl.ds(0, SB)], dstv.at[0])
    plsc.subcore_barrier()

    pltpu.async_copy(table.at[srcv.at[0]], rows.at[0], sem.at[0])

    def superblock(t, carry):
        sb = lax.rem(t, 2)

        @pl.when(t > 0)
        def _drain_cur():
            pltpu.make_async_copy(dstr.at[w, pl.ds(t * SB, SB)],
                                  dstv.at[sb], isem).wait()

        @pl.when(t + 1 < NSB)
        def _stage_next():
            pltpu.async_copy(dstr.at[w, pl.ds((t + 1) * SB, SB)],
                             dstv.at[1 - sb], isem)

        for k in range(SB):
            b = k % 2
            j = t * SB + k
            pltpu.make_async_copy(table.at[srcv.at[j]], rows.at[b],
                                  sem.at[b]).wait()

            @pl.when(j + 1 < NCHUNK)
            def _prefetch():
                pltpu.async_copy(table.at[srcv.at[j + 1]], rows.at[1 - b],
                                 sem.at[1 - b])

            pltpu.sync_copy(rows.at[b], accum.at[dstv.at[sb, k]], add=True)
            if with_cnt:
                pltpu.sync_copy(onesv, cnts.at[dstv.at[sb, k]], add=True)
        return carry

    lax.fori_loop(0, NSB, superblock, 0)
    plsc.subcore_barrier()

    pltpu.sync_copy(accum.at[sl], pout.at[c, sl])
    if with_cnt:
        pltpu.sync_copy(cnts.at[sl], cout.at[c, sl])


@functools.partial(jax.jit, static_argnames=("with_cnt",))
def _sc_agg(table, srcr, dstr, with_cnt):
    mesh = plsc.VectorSubcoreMesh(core_axis_name="c", subcore_axis_name="s")
    out_type = [jax.ShapeDtypeStruct((NC, NP, D), jnp.float32)]
    scratch = [pltpu.VMEM_SHARED((NP, D), jnp.float32)]
    if with_cnt:
        out_type.append(jax.ShapeDtypeStruct((NC, NP), jnp.float32))
        scratch.append(pltpu.VMEM_SHARED((NP,), jnp.float32))
    scratch += [
        pltpu.VMEM((NCHUNK, CHUNK), jnp.int32),
        pltpu.VMEM((2, SB, CHUNK), jnp.int32),
        pltpu.VMEM((2, CHUNK, D), jnp.float32),
    ]
    if with_cnt:
        scratch.append(pltpu.VMEM((CHUNK,), jnp.float32))
    scratch.append(pltpu.SemaphoreType.DMA((2,)))
    scratch.append(pltpu.SemaphoreType.DMA)

    inputs = [table, srcr, dstr, jnp.zeros((ROWS_PER_TILE, D), jnp.float32)]
    if with_cnt:
        inputs += [jnp.zeros((ROWS_PER_TILE,), jnp.float32),
                   jnp.ones((CHUNK,), jnp.float32)]
    f = functools.partial(
        pl.kernel,
        out_type=out_type,
        mesh=mesh,
        scratch_types=scratch,
    )(functools.partial(_sc_agg_kernel, with_cnt=with_cnt))
    return f(*inputs)


def _dense_kernel(p_ref, c_ref, x_ref, wn_ref, wr_ref, b_ref, o_ref, *, relu):
    agg = p_ref[0] + p_ref[1]
    cnt = c_ref[0, :, 0] + c_ref[1, :, 0]
    mean = agg * (1.0 / jnp.maximum(cnt, 1.0))[:, None]
    out = (jnp.dot(mean, wn_ref[...], precision=lax.Precision.HIGHEST)
           + jnp.dot(x_ref[...], wr_ref[...], precision=lax.Precision.HIGHEST)
           + b_ref[...][None, :])
    if relu:
        out = jnp.maximum(out, 0.0)
    o_ref[...] = out


def _dense(p, c, x, Wn, Wr, b, relu):
    BR = 1000
    grid = (N // BR,)
    return pl.pallas_call(
        functools.partial(_dense_kernel, relu=relu),
        grid=grid,
        in_specs=[
            pl.BlockSpec((NC, BR, D), lambda i: (0, i, 0)),
            pl.BlockSpec((NC, BR, 1), lambda i: (0, i, 0)),
            pl.BlockSpec((BR, D), lambda i: (i, 0)),
            pl.BlockSpec((D, D), lambda i: (0, 0)),
            pl.BlockSpec((D, D), lambda i: (0, 0)),
            pl.BlockSpec((D,), lambda i: (0,)),
        ],
        out_specs=pl.BlockSpec((BR, D), lambda i: (i, 0)),
        out_shape=jax.ShapeDtypeStruct((N, D), jnp.float32),
    )(p, c, x, Wn, Wr, b)


def kernel(x, edge_index, Wn1, Wr1, b1, Wn2, Wr2, b2):
    src = edge_index[0].astype(jnp.int32).reshape(NW, NCHUNK, CHUNK)
    dst = edge_index[1].astype(jnp.int32).reshape(NW, NCHUNK, CHUNK)

    p1, c1 = _sc_agg(x, src, dst, with_cnt=True)
    c1 = c1[..., None]
    h = _dense(p1, c1, x, Wn1, Wr1, b1, relu=True)
    p2 = _sc_agg(h, src, dst, with_cnt=False)
    if isinstance(p2, (list, tuple)):
        p2 = p2[0]
    out = _dense(p2, c1, h, Wn2, Wr2, b2, relu=False)
    return out


# speedup vs baseline: 3.6080x; 1.0028x over previous
import functools

import jax
import jax.numpy as jnp
from jax import lax
from jax.experimental import pallas as pl
from jax.experimental.pallas import tpu as pltpu
from jax.experimental.pallas import tpu_sc as plsc

N = 10000
E = 320000
D = 128

NC = 2
NS = 16
NW = NC * NS
CHUNK = 125
EPW = E // NW
NCHUNK = EPW // CHUNK
SB = 8
NSB = NCHUNK // SB
NP = 10240
ROWS_PER_TILE = NP // NS


def _sc_agg_kernel(*args, with_cnt):
    if with_cnt:
        (table, srcr, dstr, zrows, zcnt, ones_hbm, pout, cout,
         accum, cnts, srcv, dstv, rows, onesv, sem, ssem, csem,
         isem) = args
    else:
        (table, srcr, dstr, zrows, pout,
         accum, srcv, dstv, rows, sem, ssem, isem) = args
        csem = None
        cnts = onesv = cout = zcnt = ones_hbm = None

    c = lax.axis_index("c")
    s = lax.axis_index("s")
    w = c * NS + s
    sl = pl.ds(s * ROWS_PER_TILE, ROWS_PER_TILE)

    pltpu.sync_copy(zrows, accum.at[sl])
    if with_cnt:
        pltpu.sync_copy(zcnt, cnts.at[sl])
        pltpu.sync_copy(ones_hbm, onesv)
    pltpu.sync_copy(srcr.at[w], srcv)
    pltpu.sync_copy(dstr.at[w, pl.ds(0, SB)], dstv.at[0])
    plsc.subcore_barrier()

    pltpu.async_copy(table.at[srcv.at[0]], rows.at[0], sem.at[0])

    def superblock(t, carry):
        sb = lax.rem(t, 2)

        @pl.when(t > 0)
        def _drain_cur():
            pltpu.make_async_copy(dstr.at[w, pl.ds(t * SB, SB)],
                                  dstv.at[sb], isem).wait()

        for k in range(SB):
            b = k % 2
            j = t * SB + k
            pltpu.make_async_copy(table.at[srcv.at[j]], rows.at[b],
                                  sem.at[b]).wait()

            @pl.when(j >= 1)
            def _drain_prev_scatter():
                pltpu.make_async_copy(rows.at[1 - b],
                                      accum.at[dstv.at[sb, k]],
                                      ssem.at[1 - b]).wait()
                if with_cnt:
                    pltpu.make_async_copy(onesv, cnts.at[dstv.at[sb, k]],
                                          csem).wait()

            if k == 0:
                @pl.when(t + 1 < NSB)
                def _stage_next():
                    pltpu.async_copy(dstr.at[w, pl.ds((t + 1) * SB, SB)],
                                     dstv.at[1 - sb], isem)

            @pl.when(j + 1 < NCHUNK)
            def _prefetch():
                pltpu.async_copy(table.at[srcv.at[j + 1]], rows.at[1 - b],
                                 sem.at[1 - b])

            pltpu.async_copy(rows.at[b], accum.at[dstv.at[sb, k]],
                             ssem.at[b], add=True)
            if with_cnt:
                pltpu.async_copy(onesv, cnts.at[dstv.at[sb, k]], csem,
                                 add=True)
        return carry

    lax.fori_loop(0, NSB, superblock, 0)

    pltpu.make_async_copy(rows.at[1], accum.at[dstv.at[1, SB - 1]],
                          ssem.at[1]).wait()
    if with_cnt:
        pltpu.make_async_copy(onesv, cnts.at[dstv.at[1, SB - 1]],
                              csem).wait()
    plsc.subcore_barrier()

    pltpu.sync_copy(accum.at[sl], pout.at[c, sl])
    if with_cnt:
        pltpu.sync_copy(cnts.at[sl], cout.at[c, sl])


@functools.partial(jax.jit, static_argnames=("with_cnt",))
def _sc_agg(table, srcr, dstr, with_cnt):
    mesh = plsc.VectorSubcoreMesh(core_axis_name="c", subcore_axis_name="s")
    out_type = [jax.ShapeDtypeStruct((NC, NP, D), jnp.float32)]
    scratch = [pltpu.VMEM_SHARED((NP, D), jnp.float32)]
    if with_cnt:
        out_type.append(jax.ShapeDtypeStruct((NC, NP), jnp.float32))
        scratch.append(pltpu.VMEM_SHARED((NP,), jnp.float32))
    scratch += [
        pltpu.VMEM((NCHUNK, CHUNK), jnp.int32),
        pltpu.VMEM((2, SB, CHUNK), jnp.int32),
        pltpu.VMEM((2, CHUNK, D), jnp.float32),
    ]
    if with_cnt:
        scratch.append(pltpu.VMEM((CHUNK,), jnp.float32))
    scratch.append(pltpu.SemaphoreType.DMA((2,)))
    scratch.append(pltpu.SemaphoreType.DMA((2,)))
    if with_cnt:
        scratch.append(pltpu.SemaphoreType.DMA)
    scratch.append(pltpu.SemaphoreType.DMA)

    inputs = [table, srcr, dstr, jnp.zeros((ROWS_PER_TILE, D), jnp.float32)]
    if with_cnt:
        inputs += [jnp.zeros((ROWS_PER_TILE,), jnp.float32),
                   jnp.ones((CHUNK,), jnp.float32)]
    f = functools.partial(
        pl.kernel,
        out_type=out_type,
        mesh=mesh,
        scratch_types=scratch,
    )(functools.partial(_sc_agg_kernel, with_cnt=with_cnt))
    return f(*inputs)


def _dense_kernel(p_ref, c_ref, x_ref, wn_ref, wr_ref, b_ref, o_ref, *, relu):
    agg = p_ref[0] + p_ref[1]
    cnt = c_ref[0, :, 0] + c_ref[1, :, 0]
    mean = agg * (1.0 / jnp.maximum(cnt, 1.0))[:, None]
    out = (jnp.dot(mean, wn_ref[...], precision=lax.Precision.HIGHEST)
           + jnp.dot(x_ref[...], wr_ref[...], precision=lax.Precision.HIGHEST)
           + b_ref[...][None, :])
    if relu:
        out = jnp.maximum(out, 0.0)
    o_ref[...] = out


def _dense(p, c, x, Wn, Wr, b, relu):
    BR = 1000
    grid = (N // BR,)
    return pl.pallas_call(
        functools.partial(_dense_kernel, relu=relu),
        grid=grid,
        in_specs=[
            pl.BlockSpec((NC, BR, D), lambda i: (0, i, 0)),
            pl.BlockSpec((NC, BR, 1), lambda i: (0, i, 0)),
            pl.BlockSpec((BR, D), lambda i: (i, 0)),
            pl.BlockSpec((D, D), lambda i: (0, 0)),
            pl.BlockSpec((D, D), lambda i: (0, 0)),
            pl.BlockSpec((D,), lambda i: (0,)),
        ],
        out_specs=pl.BlockSpec((BR, D), lambda i: (i, 0)),
        out_shape=jax.ShapeDtypeStruct((N, D), jnp.float32),
    )(p, c, x, Wn, Wr, b)


def kernel(x, edge_index, Wn1, Wr1, b1, Wn2, Wr2, b2):
    src = edge_index[0].astype(jnp.int32).reshape(NW, NCHUNK, CHUNK)
    dst = edge_index[1].astype(jnp.int32).reshape(NW, NCHUNK, CHUNK)

    p1, c1 = _sc_agg(x, src, dst, with_cnt=True)
    c1 = c1[..., None]
    h = _dense(p1, c1, x, Wn1, Wr1, b1, relu=True)
    p2 = _sc_agg(h, src, dst, with_cnt=False)
    if isinstance(p2, (list, tuple)):
        p2 = p2[0]
    out = _dense(p2, c1, h, Wn2, Wr2, b2, relu=False)
    return out


# speedup vs baseline: 3.8329x; 1.0623x over previous
import functools

import jax
import jax.numpy as jnp
from jax import lax
from jax.experimental import pallas as pl
from jax.experimental.pallas import tpu as pltpu
from jax.experimental.pallas import tpu_sc as plsc

N = 10000
E = 320000
D = 128

NC = 2
NS = 16
NW = NC * NS
CHUNK = 125
EPW = E // NW
NCHUNK = EPW // CHUNK
SB = 8
NSB = NCHUNK // SB
NP = 10240
ROWS_PER_TILE = NP // NS


def _sc_agg_kernel(*args, with_cnt):
    if with_cnt:
        (table, edges, zrows, zcnt, ones_hbm, pout, cout,
         accum, cnts, srcv, dstv, rows, onesv, sem, ssem, csem,
         isem) = args
    else:
        (table, edges, zrows, pout,
         accum, srcv, dstv, rows, sem, ssem, isem) = args
        csem = None
        cnts = onesv = cout = zcnt = ones_hbm = None

    c = lax.axis_index("c")
    s = lax.axis_index("s")
    w = c * NS + s
    sl = pl.ds(s * ROWS_PER_TILE, ROWS_PER_TILE)

    pltpu.sync_copy(zrows, accum.at[sl])
    if with_cnt:
        pltpu.sync_copy(zcnt, cnts.at[sl])
        pltpu.sync_copy(ones_hbm, onesv)
    pltpu.sync_copy(edges.at[0, w], srcv)
    pltpu.sync_copy(edges.at[1, w, pl.ds(0, SB)], dstv.at[0])
    plsc.subcore_barrier()

    pltpu.async_copy(table.at[srcv.at[0]], rows.at[0], sem.at[0])

    def superblock(t, carry):
        sb = lax.rem(t, 2)

        @pl.when(t > 0)
        def _drain_cur():
            pltpu.make_async_copy(edges.at[1, w, pl.ds(t * SB, SB)],
                                  dstv.at[sb], isem).wait()

        for k in range(SB):
            b = k % 2
            j = t * SB + k
            pltpu.make_async_copy(table.at[srcv.at[j]], rows.at[b],
                                  sem.at[b]).wait()

            @pl.when(j >= 1)
            def _drain_prev_scatter():
                pltpu.make_async_copy(rows.at[1 - b],
                                      accum.at[dstv.at[sb, k]],
                                      ssem.at[1 - b]).wait()
                if with_cnt:
                    pltpu.make_async_copy(onesv, cnts.at[dstv.at[sb, k]],
                                          csem).wait()

            if k == 0:
                @pl.when(t + 1 < NSB)
                def _stage_next():
                    pltpu.async_copy(edges.at[1, w, pl.ds((t + 1) * SB, SB)],
                                     dstv.at[1 - sb], isem)

            @pl.when(j + 1 < NCHUNK)
            def _prefetch():
                pltpu.async_copy(table.at[srcv.at[j + 1]], rows.at[1 - b],
                                 sem.at[1 - b])

            pltpu.async_copy(rows.at[b], accum.at[dstv.at[sb, k]],
                             ssem.at[b], add=True)
            if with_cnt:
                pltpu.async_copy(onesv, cnts.at[dstv.at[sb, k]], csem,
                                 add=True)
        return carry

    lax.fori_loop(0, NSB, superblock, 0)

    pltpu.make_async_copy(rows.at[1], accum.at[dstv.at[1, SB - 1]],
                          ssem.at[1]).wait()
    if with_cnt:
        pltpu.make_async_copy(onesv, cnts.at[dstv.at[1, SB - 1]],
                              csem).wait()
    plsc.subcore_barrier()

    pltpu.sync_copy(accum.at[sl], pout.at[c, sl])
    if with_cnt:
        pltpu.sync_copy(cnts.at[sl], cout.at[c, sl])


@functools.partial(jax.jit, static_argnames=("with_cnt",))
def _sc_agg(table, edges, with_cnt):
    mesh = plsc.VectorSubcoreMesh(core_axis_name="c", subcore_axis_name="s")
    out_type = [jax.ShapeDtypeStruct((NC, NP, D), jnp.float32)]
    scratch = [pltpu.VMEM_SHARED((NP, D), jnp.float32)]
    if with_cnt:
        out_type.append(jax.ShapeDtypeStruct((NC, NP), jnp.float32))
        scratch.append(pltpu.VMEM_SHARED((NP,), jnp.float32))
    scratch += [
        pltpu.VMEM((NCHUNK, CHUNK), jnp.int32),
        pltpu.VMEM((2, SB, CHUNK), jnp.int32),
        pltpu.VMEM((2, CHUNK, D), jnp.float32),
    ]
    if with_cnt:
        scratch.append(pltpu.VMEM((CHUNK,), jnp.float32))
    scratch.append(pltpu.SemaphoreType.DMA((2,)))
    scratch.append(pltpu.SemaphoreType.DMA((2,)))
    if with_cnt:
        scratch.append(pltpu.SemaphoreType.DMA)
    scratch.append(pltpu.SemaphoreType.DMA)

    inputs = [table, edges, jnp.zeros((ROWS_PER_TILE, D), jnp.float32)]
    if with_cnt:
        inputs += [jnp.zeros((ROWS_PER_TILE,), jnp.float32),
                   jnp.ones((CHUNK,), jnp.float32)]
    f = functools.partial(
        pl.kernel,
        out_type=out_type,
        mesh=mesh,
        scratch_types=scratch,
    )(functools.partial(_sc_agg_kernel, with_cnt=with_cnt))
    return f(*inputs)


def _dense_kernel(p_ref, c_ref, x_ref, wn_ref, wr_ref, b_ref, o_ref, *, relu):
    agg = p_ref[0] + p_ref[1]
    cnt = c_ref[0, :, 0] + c_ref[1, :, 0]
    mean = agg * (1.0 / jnp.maximum(cnt, 1.0))[:, None]
    out = (jnp.dot(mean, wn_ref[...], precision=lax.Precision.HIGHEST)
           + jnp.dot(x_ref[...], wr_ref[...], precision=lax.Precision.HIGHEST)
           + b_ref[...][None, :])
    if relu:
        out = jnp.maximum(out, 0.0)
    o_ref[...] = out


def _dense(p, c, x, Wn, Wr, b, relu):
    BR = 2000
    grid = (N // BR,)
    return pl.pallas_call(
        functools.partial(_dense_kernel, relu=relu),
        grid=grid,
        in_specs=[
            pl.BlockSpec((NC, BR, D), lambda i: (0, i, 0)),
            pl.BlockSpec((NC, BR, 1), lambda i: (0, i, 0)),
            pl.BlockSpec((BR, D), lambda i: (i, 0)),
            pl.BlockSpec((D, D), lambda i: (0, 0)),
            pl.BlockSpec((D, D), lambda i: (0, 0)),
            pl.BlockSpec((D,), lambda i: (0,)),
        ],
        out_specs=pl.BlockSpec((BR, D), lambda i: (i, 0)),
        out_shape=jax.ShapeDtypeStruct((N, D), jnp.float32),
    )(p, c, x, Wn, Wr, b)


def kernel(x, edge_index, Wn1, Wr1, b1, Wn2, Wr2, b2):
    edges = edge_index.astype(jnp.int32).reshape(2, NW, NCHUNK, CHUNK)

    p1, c1 = _sc_agg(x, edges, with_cnt=True)
    c1 = c1[..., None]
    h = _dense(p1, c1, x, Wn1, Wr1, b1, relu=True)
    p2 = _sc_agg(h, edges, with_cnt=False)
    if isinstance(p2, (list, tuple)):
        p2 = p2[0]
    out = _dense(p2, c1, h, Wn2, Wr2, b2, relu=False)
    return out
